# baseline (speedup 1.0000x reference)
import functools

import jax
import jax.numpy as jnp
from jax import lax
from jax.experimental import pallas as pl
from jax.experimental.pallas import tpu as pltpu
from jax.experimental.pallas import tpu_sc as plsc

MAX_LENGTH = 40
EMBED_DIM = 64
BATCH = 16384
NUM_WORKERS = 32
BPW = BATCH // NUM_WORKERS
CB = 16
NCH = BPW // CB

_mesh = plsc.VectorSubcoreMesh(core_axis_name="c", subcore_axis_name="s")


@functools.partial(
    pl.kernel,
    mesh=_mesh,
    compiler_params=pltpu.CompilerParams(use_tc_tiling_on_sc=False),
    out_type=jax.ShapeDtypeStruct((BATCH, MAX_LENGTH, EMBED_DIM), jnp.float32),
    scratch_types=[
        pltpu.VMEM((BPW, MAX_LENGTH), jnp.int32),
        pltpu.VMEM((CB, MAX_LENGTH, EMBED_DIM), jnp.float32),
        pltpu.VMEM((CB, MAX_LENGTH, EMBED_DIM), jnp.float32),
        pltpu.VMEM((MAX_LENGTH, EMBED_DIM), jnp.float32),
        pltpu.SemaphoreType.DMA,
        pltpu.SemaphoreType.DMA,
        pltpu.SemaphoreType.DMA,
    ],
)
def _embed(x_hbm, tok_hbm, pos_hbm, out_hbm, idx_v, buf0, buf1, pos_v,
           sem_g0, sem_g1, sem_o):
    wid = lax.axis_index("s") * 2 + lax.axis_index("c")
    batch0 = wid * BPW
    pltpu.sync_copy(pos_hbm, pos_v)
    pltpu.sync_copy(x_hbm.at[pl.ds(batch0, BPW)], idx_v)

    bufs = (buf0, buf1)
    gsems = (sem_g0, sem_g1)

    def fire_gathers(ci, buf, sem):
        for b in range(CB):
            pltpu.async_copy(
                tok_hbm.at[idx_v.at[ci * CB + b]], buf.at[b], sem)

    def drain_gathers(buf, sem):
        pltpu.make_async_copy(out_hbm.at[pl.ds(0, CB)], buf, sem).wait()

    def wait_out(buf):
        pltpu.make_async_copy(buf, out_hbm.at[pl.ds(0, CB)], sem_o).wait()

    def add_positions(buf):
        def b_body(bb, carry):
            def l_body(l, carry2):
                for k in range(4):
                    plsc.addupdate(
                        buf.at[bb, l, pl.ds(k * 16, 16)],
                        pos_v[l, pl.ds(k * 16, 16)])
                return carry2
            return lax.fori_loop(0, MAX_LENGTH, l_body, carry)
        lax.fori_loop(0, CB, b_body, 0)

    fire_gathers(0, buf0, sem_g0)

    def pair_body(i, carry):
        for p in range(2):
            ci = 2 * i + p
            buf = bufs[p]

            @pl.when(ci >= 1)
            def _():
                wait_out(bufs[1 - p])

            @pl.when(ci + 1 < NCH)
            def _():
                fire_gathers(ci + 1, bufs[1 - p], gsems[1 - p])

            drain_gathers(buf, gsems[p])
            add_positions(buf)
            pltpu.async_copy(
                buf, out_hbm.at[pl.ds(batch0 + ci * CB, CB)], sem_o)
        return carry

    lax.fori_loop(0, NCH // 2, pair_body, 0)
    wait_out(bufs[1])


def kernel(x, token_table, position_table):
    return _embed(x, token_table, position_table)
